# baseline (speedup 1.0000x reference)
import functools

import jax
import jax.numpy as jnp
from jax import lax
from jax.experimental import pallas as pl
from jax.experimental.pallas import tpu as pltpu
from jax.experimental.pallas import tpu_sc as plsc

BATCH = 16384
D = 32
NT = 4
NC = 2
NS = 16
NW = NC * NS
BPW = BATCH // NW
CHUNK = 128
NCHUNK = BPW // CHUNK
L = 16
GROUPS = BPW // L
PITCH = 40
WIDTH = 3 * PITCH + 1

_MESH = plsc.VectorSubcoreMesh(core_axis_name="c", subcore_axis_name="s")


@functools.partial(
    pl.kernel,
    out_type=jax.ShapeDtypeStruct((BATCH, NT * D), jnp.float32),
    mesh=_MESH,
    compiler_params=pltpu.CompilerParams(
        use_tc_tiling_on_sc=False, needs_layout_passes=False),
    scratch_types=[
        pltpu.VMEM((NT, BPW), jnp.int32),
        pltpu.VMEM((2 * L, WIDTH), jnp.float32),
        pltpu.VMEM((BPW, D), jnp.float32),
        pltpu.VMEM((BPW, WIDTH), jnp.float32),
        pltpu.SemaphoreType.DMA,
        pltpu.SemaphoreType.DMA,
    ],
)
def _emb_kernel(g_idx, a_idx, o_idx, z_idx, w_gender, w_age, w_occ, w_area,
                out_hbm, idx_v, small_v, area_v, big_v, gsem, osem):
    wid = lax.axis_index("s") * NC + lax.axis_index("c")
    base = wid * BPW
    for t, arr in enumerate((g_idx, a_idx, o_idx, z_idx)):
        pltpu.sync_copy(arr.at[pl.ds(base, BPW)], idx_v.at[t])
    copies = [
        pltpu.async_copy(
            w_area.at[idx_v.at[3, pl.ds(j * CHUNK, CHUNK)]],
            area_v.at[pl.ds(j * CHUNK, CHUNK)],
            gsem,
        )
        for j in range(NCHUNK)
    ]
    for t, w in enumerate((w_gender, w_age, w_occ)):
        pltpu.sync_copy(w, small_v.at[pl.ds(0, w.shape[0]),
                                      pl.ds(t * PITCH, D)])

    lane = lax.broadcasted_iota(jnp.int32, (L,), 0)

    def group_body(i, carry):
        rbase = i * L
        rows = rbase + lane
        for t in range(NT - 1):
            ridx = idx_v[t, pl.ds(rbase, L)]
            for c in range(D):
                col = jnp.full((L,), t * PITCH + c, jnp.int32)
                val = plsc.load_gather(small_v, [ridx, col])
                plsc.store_scatter(big_v, [rows, col], val)
        return carry

    lax.fori_loop(0, GROUPS, group_body, 0)
    writes = [
        pltpu.async_copy(
            big_v.at[pl.ds(0, BPW), pl.ds(t * PITCH, D)],
            out_hbm.at[pl.ds(base, BPW), pl.ds(t * D, D)],
            osem,
        )
        for t in range(NT - 1)
    ]
    for cpy in copies:
        cpy.wait()
    writes.append(pltpu.async_copy(
        area_v, out_hbm.at[pl.ds(base, BPW), pl.ds((NT - 1) * D, D)], osem))
    for wr in writes:
        wr.wait()


def kernel(gender_idx, age_idx, occupation_idx, area_idx,
           W_gender, W_age, W_occupation, W_area):
    return _emb_kernel(
        gender_idx.astype(jnp.int32), age_idx.astype(jnp.int32),
        occupation_idx.astype(jnp.int32), area_idx.astype(jnp.int32),
        W_gender, W_age, W_occupation, W_area)


# speedup vs baseline: 1.5148x; 1.5148x over previous
import functools

import jax
import jax.numpy as jnp
from jax import lax
from jax.experimental import pallas as pl
from jax.experimental.pallas import tpu as pltpu
from jax.experimental.pallas import tpu_sc as plsc

BATCH = 16384
D = 32
NT = 4
NC = 2
NS = 16
NW = NC * NS
BPW = BATCH // NW
CHUNK = 128
NCHUNK = BPW // CHUNK
L = 16
GROUPS = BPW // L
PITCH = 40
WIDTH = 3 * PITCH

_MESH = plsc.VectorSubcoreMesh(core_axis_name="c", subcore_axis_name="s")


@functools.partial(
    pl.kernel,
    out_type=jax.ShapeDtypeStruct((BATCH, NT * D), jnp.float32),
    mesh=_MESH,
    compiler_params=pltpu.CompilerParams(
        use_tc_tiling_on_sc=False, needs_layout_passes=False),
    scratch_types=[
        pltpu.VMEM((NT, BPW), jnp.int32),
        pltpu.VMEM((2 * L, WIDTH), jnp.float32),
        pltpu.VMEM((BPW, D), jnp.float32),
        pltpu.VMEM((BPW, WIDTH), jnp.float32),
        pltpu.SemaphoreType.DMA,
        pltpu.SemaphoreType.DMA,
        pltpu.SemaphoreType.DMA,
    ],
)
def _emb_kernel(g_idx, a_idx, o_idx, z_idx, w_gender, w_age, w_occ, w_area,
                out_hbm, idx_v, small_v, area_v, big_v, gsem, osem, isem):
    wid = lax.axis_index("s") * NC + lax.axis_index("c")
    base = wid * BPW
    cz = pltpu.async_copy(z_idx.at[pl.ds(base, BPW)], idx_v.at[3], isem)
    cidx = [pltpu.async_copy(arr.at[pl.ds(base, BPW)], idx_v.at[t], isem)
            for t, arr in enumerate((g_idx, a_idx, o_idx))]
    for t, w in enumerate((w_gender, w_age, w_occ)):
        cidx.append(pltpu.async_copy(
            w, small_v.at[pl.ds(0, w.shape[0]), pl.ds(t * PITCH, D)], isem))
    cz.wait()
    copies = [
        pltpu.async_copy(
            w_area.at[idx_v.at[3, pl.ds(j * CHUNK, CHUNK)]],
            area_v.at[pl.ds(j * CHUNK, CHUNK)],
            gsem,
        )
        for j in range(NCHUNK)
    ]
    for c in cidx:
        c.wait()

    def group_body(i, carry):
        rbase = i * L
        for t in range(NT - 1):
            ridx = idx_v[t, pl.ds(rbase, L)]
            for j in range(L):
                s = ridx[j]
                r = rbase + j
                for h in (0, L):
                    big_v[r, pl.ds(t * PITCH + h, L)] = (
                        small_v[s, pl.ds(t * PITCH + h, L)])
        return carry

    lax.fori_loop(0, GROUPS, group_body, 0)
    writes = [
        pltpu.async_copy(
            big_v.at[pl.ds(0, BPW), pl.ds(t * PITCH, D)],
            out_hbm.at[pl.ds(base, BPW), pl.ds(t * D, D)],
            osem,
        )
        for t in range(NT - 1)
    ]
    for cpy in copies:
        cpy.wait()
    writes.append(pltpu.async_copy(
        area_v, out_hbm.at[pl.ds(base, BPW), pl.ds((NT - 1) * D, D)], osem))
    for wr in writes:
        wr.wait()


def kernel(gender_idx, age_idx, occupation_idx, area_idx,
           W_gender, W_age, W_occupation, W_area):
    return _emb_kernel(
        gender_idx.astype(jnp.int32), age_idx.astype(jnp.int32),
        occupation_idx.astype(jnp.int32), area_idx.astype(jnp.int32),
        W_gender, W_age, W_occupation, W_area)


# speedup vs baseline: 1.7888x; 1.1809x over previous
import functools

import jax
import jax.numpy as jnp
from jax import lax
from jax.experimental import pallas as pl
from jax.experimental.pallas import tpu as pltpu
from jax.experimental.pallas import tpu_sc as plsc

BATCH = 16384
D = 32
NT = 4
NC = 2
NS = 16
NW = NC * NS
BPW = BATCH // NW
CHUNK = 128
NCHUNK = BPW // CHUNK
L = 16
GROUPS = BPW // L
SW = (NT - 1) * D

_MESH = plsc.VectorSubcoreMesh(core_axis_name="c", subcore_axis_name="s")
_PARAMS = pltpu.CompilerParams(
    use_tc_tiling_on_sc=False, needs_layout_passes=False)


@functools.partial(
    pl.kernel,
    out_type=jax.ShapeDtypeStruct((BATCH, SW), jnp.float32),
    mesh=_MESH,
    compiler_params=_PARAMS,
    scratch_types=[
        pltpu.VMEM((NT - 1, BPW), jnp.int32),
        pltpu.VMEM((2 * L, D), jnp.float32),
        pltpu.VMEM((2 * L, D), jnp.float32),
        pltpu.VMEM((2 * L, D), jnp.float32),
        pltpu.VMEM((BPW, SW), jnp.float32),
        pltpu.SemaphoreType.DMA,
    ],
)
def _small_kernel(g_idx, a_idx, o_idx, w_gender, w_age, w_occ, out_hbm,
                  idx_v, sg_v, sa_v, so_v, big_v, isem):
    wid = lax.axis_index("s") * NC + lax.axis_index("c")
    base = wid * BPW
    stage = [pltpu.async_copy(arr.at[pl.ds(base, BPW)], idx_v.at[t], isem)
             for t, arr in enumerate((g_idx, a_idx, o_idx))]
    for w, buf in ((w_gender, sg_v), (w_age, sa_v), (w_occ, so_v)):
        stage.append(pltpu.async_copy(
            w, buf.at[pl.ds(0, w.shape[0])], isem))
    for c in stage:
        c.wait()

    def group_body(i, carry):
        rbase = i * L
        for t, buf in ((0, sg_v), (1, sa_v), (2, so_v)):
            ridx = idx_v[t, pl.ds(rbase, L)]
            for j in range(L):
                s = ridx[j]
                r = rbase + j
                for h in (0, L):
                    big_v[r, pl.ds(t * D + h, L)] = buf[s, pl.ds(h, L)]
        return carry

    lax.fori_loop(0, GROUPS, group_body, 0)
    pltpu.sync_copy(big_v, out_hbm.at[pl.ds(base, BPW)])


@functools.partial(
    pl.kernel,
    out_type=jax.ShapeDtypeStruct((BATCH, NT * D), jnp.float32),
    mesh=_MESH,
    compiler_params=_PARAMS,
    scratch_types=[
        pltpu.VMEM((BPW,), jnp.int32),
        pltpu.VMEM((BPW, D), jnp.float32),
        pltpu.VMEM((BPW, SW), jnp.float32),
        pltpu.SemaphoreType.DMA,
        pltpu.SemaphoreType.DMA,
    ],
)
def _area_kernel(z_idx, w_area, small_hbm, out_hbm,
                 idx_v, area_v, sm_v, gsem, osem):
    wid = lax.axis_index("s") * NC + lax.axis_index("c")
    base = wid * BPW
    pltpu.sync_copy(z_idx.at[pl.ds(base, BPW)], idx_v)
    copies = [
        pltpu.async_copy(
            w_area.at[idx_v.at[pl.ds(j * CHUNK, CHUNK)]],
            area_v.at[pl.ds(j * CHUNK, CHUNK)],
            gsem,
        )
        for j in range(NCHUNK)
    ]
    pltpu.sync_copy(small_hbm.at[pl.ds(base, BPW)], sm_v)
    wr_small = pltpu.async_copy(
        sm_v, out_hbm.at[pl.ds(base, BPW), pl.ds(0, SW)], osem)
    for c in copies:
        c.wait()
    wr_area = pltpu.async_copy(
        area_v, out_hbm.at[pl.ds(base, BPW), pl.ds(SW, D)], osem)
    wr_small.wait()
    wr_area.wait()


def kernel(gender_idx, age_idx, occupation_idx, area_idx,
           W_gender, W_age, W_occupation, W_area):
    small = _small_kernel(
        gender_idx.astype(jnp.int32), age_idx.astype(jnp.int32),
        occupation_idx.astype(jnp.int32), W_gender, W_age, W_occupation)
    return _area_kernel(area_idx.astype(jnp.int32), W_area, small)
